# baseline (speedup 1.0000x reference)
import functools

import jax
import jax.numpy as jnp
from jax import lax
from jax.experimental import pallas as pl
from jax.experimental.pallas import tpu as pltpu
from jax.experimental.pallas import tpu_sc as plsc

_TEMP = 0.1
_K = 4
_IDXBITS = 0x1FFF
_IMAX = 0x7FFFFFFF


def _norm_body(x_ref, o_ref):
    x = x_ref[...]
    n2 = jnp.sum(x * x, axis=1, keepdims=True)
    o_ref[...] = x * lax.rsqrt(jnp.maximum(n2, 1e-24))


def _normalize(x, bn=1024):
    n, c = x.shape
    return pl.pallas_call(
        _norm_body,
        grid=(n // bn,),
        in_specs=[pl.BlockSpec((bn, c), lambda i: (i, 0))],
        out_specs=pl.BlockSpec((bn, c), lambda i: (i, 0)),
        out_shape=jax.ShapeDtypeStruct((n, c), jnp.float32),
    )(x)


def _knn_body(ca_ref, cbt_ref, idx_ref, r2_ref):
    ca = ca_ref[...]
    cbt = cbt_ref[...]

    @pl.when(pl.program_id(0) == 0)
    def _prep():
        r2_ref[...] = jnp.sum(cbt * cbt, axis=0, keepdims=True)

    q2 = jnp.sum(ca * ca, axis=1, keepdims=True)
    cross = jnp.dot(ca, cbt, preferred_element_type=jnp.float32)
    d2 = jnp.maximum(q2 - 2.0 * cross + r2_ref[...], 0.0)

    col = lax.broadcasted_iota(jnp.int32, d2.shape, 1)
    key = (pltpu.bitcast(d2, jnp.int32) & ~_IDXBITS) | col

    picks = []
    for _ in range(_K):
        m = jnp.min(key, axis=1, keepdims=True)
        key = jnp.where(key == m, _IMAX, key)
        picks.append(m & _IDXBITS)
    idx_ref[...] = jnp.concatenate(picks, axis=1)


def _knn(coord_a, coord_b, bq=512):
    na = coord_a.shape[0]
    nb = coord_b.shape[0]
    ca = jnp.pad(coord_a, ((0, 0), (0, 5)))
    cbt = jnp.pad(coord_b, ((0, 0), (0, 5))).T
    return pl.pallas_call(
        _knn_body,
        grid=(na // bq,),
        in_specs=[
            pl.BlockSpec((bq, 8), lambda i: (i, 0)),
            pl.BlockSpec((8, nb), lambda i: (0, 0)),
        ],
        out_specs=pl.BlockSpec((bq, _K), lambda i: (i, 0)),
        out_shape=jax.ShapeDtypeStruct((na, _K), jnp.int32),
        scratch_shapes=[pltpu.VMEM((1, nb), jnp.float32)],
        compiler_params=pltpu.CompilerParams(
            dimension_semantics=("arbitrary",)),
    )(ca, cbt)


_NC = 2
_NS = 16
_NW = _NC * _NS
_SUB = 16


def _gdot_body(qpw, c, table_ref, idx_ref, fq_ref,
               o0_ref, o1_ref, o2_ref, o3_ref,
               idx_v, fa_v, rows0_v, rows1_v, pacc_v, dots_v, sem0, sem1):
    wid = lax.axis_index("s") * _NC + lax.axis_index("c")
    qw = wid * qpw
    outs = (o0_ref, o1_ref, o2_ref, o3_ref)
    lane = lax.iota(jnp.int32, 16)
    nsub = qpw // _SUB
    rows = (rows0_v, rows1_v)
    sems = (sem0, sem1)

    pltpu.sync_copy(idx_ref.at[pl.ds(qw * _K, qpw * _K)], idx_v)
    pltpu.sync_copy(fq_ref.at[pl.ds(qw, qpw)], fa_v)

    def fire(sb):
        b = sb % 2
        pltpu.async_copy(
            table_ref.at[idx_v.at[pl.ds(sb * _SUB * _K, _SUB * _K)]],
            rows[b], sems[b])

    def drain(sb):
        b = sb % 2
        pltpu.make_async_copy(
            table_ref.at[idx_v.at[pl.ds(sb * _SUB * _K, _SUB * _K)]],
            rows[b], sems[b]).wait()

    fire(0)
    for sb in range(nsub):
        drain(sb)
        if sb + 1 < nsub:
            fire(sb + 1)
        rv = rows[sb % 2]

        def qbody(q, _, _rv=rv, _sb=sb):
            accs = [jnp.zeros((16,), jnp.float32) for _ in range(_K)]
            for c16 in range(c // 16):
                sl = pl.ds(c16 * 16, 16)
                fav = fa_v[_sb * _SUB + q, sl]
                for k in range(_K):
                    accs[k] = accs[k] + fav * _rv[q * _K + k, sl]
            for k in range(_K):
                pacc_v[k * 16 + q, pl.ds(0, 16)] = accs[k]
            return 0

        lax.fori_loop(0, _SUB, qbody, 0, unroll=2)
        for k in range(_K):
            rid = lane + k * 16
            tot = jnp.zeros((16,), jnp.float32)
            for j in range(16):
                tot = tot + plsc.load_gather(pacc_v, [rid, lane * 0 + j])
            dots_v[k, pl.ds(sb * _SUB, _SUB)] = tot
    for k in range(_K):
        pltpu.sync_copy(dots_v.at[k], outs[k].at[pl.ds(qw, qpw)])


def _gather_dot(table, idx_flat, fq):
    na, c = fq.shape
    qpw = na // _NW
    mesh = plsc.VectorSubcoreMesh(core_axis_name="c", subcore_axis_name="s")
    out = jax.ShapeDtypeStruct((na,), jnp.float32)
    f = pl.kernel(
        functools.partial(_gdot_body, qpw, c),
        out_type=(out, out, out, out),
        mesh=mesh,
        scratch_types=[
            pltpu.VMEM((qpw * _K,), jnp.int32),
            pltpu.VMEM((qpw, c), jnp.float32),
            pltpu.VMEM((_SUB * _K, c), jnp.float32),
            pltpu.VMEM((_SUB * _K, c), jnp.float32),
            pltpu.VMEM((_K * 16, 17), jnp.float32),
            pltpu.VMEM((_K, qpw), jnp.float32),
            pltpu.SemaphoreType.DMA,
            pltpu.SemaphoreType.DMA,
        ],
        compiler_params=pltpu.CompilerParams(needs_layout_passes=False),
    )
    return f(table, idx_flat, fq)


def _loss_body(d0_ref, d1_ref, d2_ref, d3_ref, out_ref):
    ds = [d0_ref[...], d1_ref[...], d2_ref[...], d3_ref[...]]
    ls = [d * (1.0 / _TEMP) for d in ds]
    mx = jnp.maximum(jnp.maximum(ls[0], ls[1]), jnp.maximum(ls[2], ls[3]))
    es = [jnp.exp(l - mx) for l in ls]
    z = es[0] + es[1] + es[2] + es[3]
    p2 = (es[0] * es[0] + es[1] * es[1] + es[2] * es[2] + es[3] * es[3]) / (z * z)
    out_ref[0, 0] = jnp.sum(-jnp.log(p2 + 1e-12))


def _loss_sum(dots):
    n = dots[0].shape[0]
    rows = n // 128
    ds2 = [d.reshape(rows, 128) for d in dots]
    spec = pl.BlockSpec((rows, 128), lambda: (0, 0))
    out = pl.pallas_call(
        _loss_body,
        in_specs=[spec] * 4,
        out_specs=pl.BlockSpec(memory_space=pltpu.SMEM),
        out_shape=jax.ShapeDtypeStruct((1, 1), jnp.float32),
    )(*ds2)
    return out[0, 0]


def kernel(feat_0, coord_0, feat_1, coord_1):
    n0 = feat_0.shape[0]
    n1 = feat_1.shape[0]
    f0n = _normalize(feat_0)
    f1n = _normalize(feat_1)
    idx0 = _knn(coord_0, coord_1)
    idx1 = _knn(coord_1, coord_0)
    dots0 = _gather_dot(f1n, idx0.reshape(-1), f0n)
    dots1 = _gather_dot(f0n, idx1.reshape(-1), f1n)
    loss0 = _loss_sum(dots0) / n0
    loss1 = _loss_sum(dots1) / n1
    return 0.5 * (loss0 + loss1)


# speedup vs baseline: 1.0073x; 1.0073x over previous
import functools

import jax
import jax.numpy as jnp
from jax import lax
from jax.experimental import pallas as pl
from jax.experimental.pallas import tpu as pltpu
from jax.experimental.pallas import tpu_sc as plsc

_TEMP = 0.1
_K = 4
_IDXBITS = 0x1FFF
_IMAX = 0x7FFFFFFF


def _norm_body(x_ref, o_ref):
    x = x_ref[...]
    n2 = jnp.sum(x * x, axis=1, keepdims=True)
    o_ref[...] = x * lax.rsqrt(jnp.maximum(n2, 1e-24))


def _normalize(x, bn=1024):
    n, c = x.shape
    return pl.pallas_call(
        _norm_body,
        grid=(n // bn,),
        in_specs=[pl.BlockSpec((bn, c), lambda i: (i, 0))],
        out_specs=pl.BlockSpec((bn, c), lambda i: (i, 0)),
        out_shape=jax.ShapeDtypeStruct((n, c), jnp.float32),
    )(x)


def _knn_body(ca_ref, cbt_ref, idx_ref, r2_ref):
    ca = ca_ref[...]
    cbt = cbt_ref[...]

    @pl.when(pl.program_id(0) == 0)
    def _prep():
        r2_ref[...] = jnp.sum(cbt * cbt, axis=0, keepdims=True)

    q2 = jnp.sum(ca * ca, axis=1, keepdims=True)
    cross = jnp.dot(ca, cbt, preferred_element_type=jnp.float32)
    d2 = jnp.maximum(q2 - 2.0 * cross + r2_ref[...], 0.0)

    col = lax.broadcasted_iota(jnp.int32, d2.shape, 1)
    key = (pltpu.bitcast(d2, jnp.int32) & ~_IDXBITS) | col

    picks = []
    for r in range(_K):
        m = jnp.min(key, axis=1, keepdims=True)
        if r + 1 < _K:
            key = jnp.where(key == m, _IMAX, key)
        picks.append(m & _IDXBITS)
    idx_ref[...] = jnp.concatenate(picks, axis=1)


def _knn(coord_a, coord_b, bq=512):
    na = coord_a.shape[0]
    nb = coord_b.shape[0]
    ca = jnp.pad(coord_a, ((0, 0), (0, 5)))
    cbt = jnp.pad(coord_b, ((0, 0), (0, 5))).T
    return pl.pallas_call(
        _knn_body,
        grid=(na // bq,),
        in_specs=[
            pl.BlockSpec((bq, 8), lambda i: (i, 0)),
            pl.BlockSpec((8, nb), lambda i: (0, 0)),
        ],
        out_specs=pl.BlockSpec((bq, _K), lambda i: (i, 0)),
        out_shape=jax.ShapeDtypeStruct((na, _K), jnp.int32),
        scratch_shapes=[pltpu.VMEM((1, nb), jnp.float32)],
        compiler_params=pltpu.CompilerParams(
            dimension_semantics=("arbitrary",)),
    )(ca, cbt)


_NC = 2
_NS = 16
_NW = _NC * _NS
_SUB = 16


def _gdot_body(qpw, c, table_ref, idx_ref, fq_ref,
               o0_ref, o1_ref, o2_ref, o3_ref,
               idx_v, fa_v, rows0_v, rows1_v, pacc_v, dots_v, sem0, sem1):
    wid = lax.axis_index("s") * _NC + lax.axis_index("c")
    qw = wid * qpw
    outs = (o0_ref, o1_ref, o2_ref, o3_ref)
    lane = lax.iota(jnp.int32, 16)
    nsub = qpw // _SUB
    rows = (rows0_v, rows1_v)
    sems = (sem0, sem1)

    pltpu.sync_copy(idx_ref.at[pl.ds(qw * _K, qpw * _K)], idx_v)
    pltpu.sync_copy(fq_ref.at[pl.ds(qw, qpw)], fa_v)

    def fire(sb):
        b = sb % 2
        pltpu.async_copy(
            table_ref.at[idx_v.at[pl.ds(sb * _SUB * _K, _SUB * _K)]],
            rows[b], sems[b])

    def drain(sb):
        b = sb % 2
        pltpu.make_async_copy(
            table_ref.at[idx_v.at[pl.ds(sb * _SUB * _K, _SUB * _K)]],
            rows[b], sems[b]).wait()

    fire(0)
    for sb in range(nsub):
        drain(sb)
        if sb + 1 < nsub:
            fire(sb + 1)
        rv = rows[sb % 2]

        def qbody(q, _, _rv=rv, _sb=sb):
            accs = [jnp.zeros((16,), jnp.float32) for _ in range(_K)]
            for c16 in range(c // 16):
                sl = pl.ds(c16 * 16, 16)
                fav = fa_v[_sb * _SUB + q, sl]
                for k in range(_K):
                    accs[k] = accs[k] + fav * _rv[q * _K + k, sl]
            for k in range(_K):
                pacc_v[k * 16 + q, pl.ds(0, 16)] = accs[k]
            return 0

        lax.fori_loop(0, _SUB, qbody, 0, unroll=False)
        for k in range(_K):
            rid = lane + k * 16
            tot = jnp.zeros((16,), jnp.float32)
            for j in range(16):
                tot = tot + plsc.load_gather(pacc_v, [rid, lane * 0 + j])
            dots_v[k, pl.ds(sb * _SUB, _SUB)] = tot
    for k in range(_K):
        pltpu.sync_copy(dots_v.at[k], outs[k].at[pl.ds(qw, qpw)])


def _gather_dot(table, idx_flat, fq):
    na, c = fq.shape
    qpw = na // _NW
    mesh = plsc.VectorSubcoreMesh(core_axis_name="c", subcore_axis_name="s")
    out = jax.ShapeDtypeStruct((na,), jnp.float32)
    f = pl.kernel(
        functools.partial(_gdot_body, qpw, c),
        out_type=(out, out, out, out),
        mesh=mesh,
        scratch_types=[
            pltpu.VMEM((qpw * _K,), jnp.int32),
            pltpu.VMEM((qpw, c), jnp.float32),
            pltpu.VMEM((_SUB * _K, c), jnp.float32),
            pltpu.VMEM((_SUB * _K, c), jnp.float32),
            pltpu.VMEM((_K * 16, 17), jnp.float32),
            pltpu.VMEM((_K, qpw), jnp.float32),
            pltpu.SemaphoreType.DMA,
            pltpu.SemaphoreType.DMA,
        ],
        compiler_params=pltpu.CompilerParams(needs_layout_passes=False),
    )
    return f(table, idx_flat, fq)


def _loss_body(d0_ref, d1_ref, d2_ref, d3_ref, out_ref):
    ds = [d0_ref[...], d1_ref[...], d2_ref[...], d3_ref[...]]
    ls = [d * (1.0 / _TEMP) for d in ds]
    mx = jnp.maximum(jnp.maximum(ls[0], ls[1]), jnp.maximum(ls[2], ls[3]))
    es = [jnp.exp(l - mx) for l in ls]
    z = es[0] + es[1] + es[2] + es[3]
    p2 = (es[0] * es[0] + es[1] * es[1] + es[2] * es[2] + es[3] * es[3]) / (z * z)
    out_ref[0, 0] = jnp.sum(-jnp.log(p2 + 1e-12))


def _loss_sum(dots):
    n = dots[0].shape[0]
    rows = n // 128
    ds2 = [d.reshape(rows, 128) for d in dots]
    spec = pl.BlockSpec((rows, 128), lambda: (0, 0))
    out = pl.pallas_call(
        _loss_body,
        in_specs=[spec] * 4,
        out_specs=pl.BlockSpec(memory_space=pltpu.SMEM),
        out_shape=jax.ShapeDtypeStruct((1, 1), jnp.float32),
    )(*ds2)
    return out[0, 0]


def kernel(feat_0, coord_0, feat_1, coord_1):
    n0 = feat_0.shape[0]
    n1 = feat_1.shape[0]
    f0n = _normalize(feat_0)
    f1n = _normalize(feat_1)
    idx0 = _knn(coord_0, coord_1)
    idx1 = _knn(coord_1, coord_0)
    dots0 = _gather_dot(f1n, idx0.reshape(-1), f0n)
    dots1 = _gather_dot(f0n, idx1.reshape(-1), f1n)
    loss0 = _loss_sum(dots0) / n0
    loss1 = _loss_sum(dots1) / n1
    return 0.5 * (loss0 + loss1)


# speedup vs baseline: 1.0075x; 1.0003x over previous
import functools

import jax
import jax.numpy as jnp
from jax import lax
from jax.experimental import pallas as pl
from jax.experimental.pallas import tpu as pltpu
from jax.experimental.pallas import tpu_sc as plsc

_TEMP = 0.1
_K = 4
_IDXBITS = 0x1FFF
_IMAX = 0x7FFFFFFF


def _norm_body(x_ref, o_ref):
    x = x_ref[...]
    n2 = jnp.sum(x * x, axis=1, keepdims=True)
    o_ref[...] = x * lax.rsqrt(jnp.maximum(n2, 1e-24))


def _normalize(x, bn=1024):
    n, c = x.shape
    return pl.pallas_call(
        _norm_body,
        grid=(n // bn,),
        in_specs=[pl.BlockSpec((bn, c), lambda i: (i, 0))],
        out_specs=pl.BlockSpec((bn, c), lambda i: (i, 0)),
        out_shape=jax.ShapeDtypeStruct((n, c), jnp.float32),
    )(x)


def _knn_body(idxbits, ca_ref, cbt_ref, idx_ref, r2_ref):
    ca = ca_ref[...]
    cbt = cbt_ref[...]

    @pl.when(pl.program_id(0) == 0)
    def _prep():
        r2_ref[...] = jnp.sum(cbt * cbt, axis=0, keepdims=True)

    q2 = jnp.sum(ca * ca, axis=1, keepdims=True)
    cross = jnp.dot(ca, cbt, preferred_element_type=jnp.float32)
    d2 = jnp.maximum(q2 - 2.0 * cross + r2_ref[...], 0.0)

    col = lax.broadcasted_iota(jnp.int32, d2.shape, 1)
    key = (pltpu.bitcast(d2, jnp.int32) & ~idxbits) | col

    picks = []
    for r in range(_K):
        m = jnp.min(key, axis=1, keepdims=True)
        if r + 1 < _K:
            key = jnp.where(key == m, _IMAX, key)
        picks.append(m & idxbits)
    idx_ref[...] = jnp.concatenate(picks, axis=1)


def _knn(coord_a, coord_b, bq=512):
    na = coord_a.shape[0]
    nb = coord_b.shape[0]
    idxbits = (1 << max(12, (nb - 1).bit_length())) - 1
    ca = jnp.pad(coord_a, ((0, 0), (0, 5)))
    cbt = jnp.pad(coord_b, ((0, 0), (0, 5))).T
    return pl.pallas_call(
        functools.partial(_knn_body, idxbits),
        grid=(na // bq,),
        in_specs=[
            pl.BlockSpec((bq, 8), lambda i: (i, 0)),
            pl.BlockSpec((8, nb), lambda i: (0, 0)),
        ],
        out_specs=pl.BlockSpec((bq, _K), lambda i: (i, 0)),
        out_shape=jax.ShapeDtypeStruct((na, _K), jnp.int32),
        scratch_shapes=[pltpu.VMEM((1, nb), jnp.float32)],
        compiler_params=pltpu.CompilerParams(
            dimension_semantics=("arbitrary",)),
    )(ca, cbt)


_NC = 2
_NS = 16
_NW = _NC * _NS
_SUB = 16


def _gdot_body(qpw, c, table_ref, idx_ref, fq_ref,
               o0_ref, o1_ref, o2_ref, o3_ref,
               idx_v, fa_v, rows0_v, rows1_v, pacc_v, dots_v, sem0, sem1):
    wid = lax.axis_index("s") * _NC + lax.axis_index("c")
    qw = wid * qpw
    outs = (o0_ref, o1_ref, o2_ref, o3_ref)
    lane = lax.iota(jnp.int32, 16)
    nsub = qpw // _SUB
    rows = (rows0_v, rows1_v)
    sems = (sem0, sem1)

    pltpu.sync_copy(idx_ref.at[pl.ds(qw * _K, qpw * _K)], idx_v)
    pltpu.sync_copy(fq_ref.at[pl.ds(qw, qpw)], fa_v)

    def fire(sb):
        b = sb % 2
        pltpu.async_copy(
            table_ref.at[idx_v.at[pl.ds(sb * _SUB * _K, _SUB * _K)]],
            rows[b], sems[b])

    def drain(sb):
        b = sb % 2
        pltpu.make_async_copy(
            table_ref.at[idx_v.at[pl.ds(sb * _SUB * _K, _SUB * _K)]],
            rows[b], sems[b]).wait()

    fire(0)
    for sb in range(nsub):
        drain(sb)
        if sb + 1 < nsub:
            fire(sb + 1)
        rv = rows[sb % 2]

        def qbody(q, _, _rv=rv, _sb=sb):
            accs = [jnp.zeros((16,), jnp.float32) for _ in range(_K)]
            for c16 in range(c // 16):
                sl = pl.ds(c16 * 16, 16)
                fav = fa_v[_sb * _SUB + q, sl]
                for k in range(_K):
                    accs[k] = accs[k] + fav * _rv[q * _K + k, sl]
            for k in range(_K):
                pacc_v[k * 16 + q, pl.ds(0, 16)] = accs[k]
            return 0

        lax.fori_loop(0, _SUB, qbody, 0, unroll=False)
        for k in range(_K):
            rid = lane + k * 16
            tot = jnp.zeros((16,), jnp.float32)
            for j in range(16):
                tot = tot + plsc.load_gather(pacc_v, [rid, lane * 0 + j])
            dots_v[k, pl.ds(sb * _SUB, _SUB)] = tot
    for k in range(_K):
        pltpu.sync_copy(dots_v.at[k], outs[k].at[pl.ds(qw, qpw)])


def _gather_dot(table, idx_flat, fq):
    na, c = fq.shape
    qpw = na // _NW
    mesh = plsc.VectorSubcoreMesh(core_axis_name="c", subcore_axis_name="s")
    out = jax.ShapeDtypeStruct((na,), jnp.float32)
    f = pl.kernel(
        functools.partial(_gdot_body, qpw, c),
        out_type=(out, out, out, out),
        mesh=mesh,
        scratch_types=[
            pltpu.VMEM((qpw * _K,), jnp.int32),
            pltpu.VMEM((qpw, c), jnp.float32),
            pltpu.VMEM((_SUB * _K, c), jnp.float32),
            pltpu.VMEM((_SUB * _K, c), jnp.float32),
            pltpu.VMEM((_K * 16, 17), jnp.float32),
            pltpu.VMEM((_K, qpw), jnp.float32),
            pltpu.SemaphoreType.DMA,
            pltpu.SemaphoreType.DMA,
        ],
        compiler_params=pltpu.CompilerParams(needs_layout_passes=False),
    )
    return f(table, idx_flat, fq)


def _loss_body(d0_ref, d1_ref, d2_ref, d3_ref, out_ref):
    ds = [d0_ref[...], d1_ref[...], d2_ref[...], d3_ref[...]]
    ls = [d * (1.0 / _TEMP) for d in ds]
    mx = jnp.maximum(jnp.maximum(ls[0], ls[1]), jnp.maximum(ls[2], ls[3]))
    es = [jnp.exp(l - mx) for l in ls]
    z = es[0] + es[1] + es[2] + es[3]
    p2 = (es[0] * es[0] + es[1] * es[1] + es[2] * es[2] + es[3] * es[3]) / (z * z)
    out_ref[0, 0] = jnp.sum(-jnp.log(p2 + 1e-12))


def _loss_sum(dots):
    n = dots[0].shape[0]
    rows = n // 128
    ds2 = [d.reshape(rows, 128) for d in dots]
    spec = pl.BlockSpec((rows, 128), lambda: (0, 0))
    out = pl.pallas_call(
        _loss_body,
        in_specs=[spec] * 4,
        out_specs=pl.BlockSpec(memory_space=pltpu.SMEM),
        out_shape=jax.ShapeDtypeStruct((1, 1), jnp.float32),
    )(*ds2)
    return out[0, 0]


def kernel(feat_0, coord_0, feat_1, coord_1):
    n0 = feat_0.shape[0]
    n1 = feat_1.shape[0]
    f0n = _normalize(feat_0)
    f1n = _normalize(feat_1)
    idx0 = _knn(coord_0, coord_1)
    idx1 = _knn(coord_1, coord_0)
    dots0 = _gather_dot(f1n, idx0.reshape(-1), f0n)
    dots1 = _gather_dot(f0n, idx1.reshape(-1), f1n)
    loss0 = _loss_sum(dots0) / n0
    loss1 = _loss_sum(dots1) / n1
    return 0.5 * (loss0 + loss1)
